# baseline (speedup 1.0000x reference)
import jax
import jax.numpy as jnp
from jax import lax
from jax.experimental import pallas as pl
from jax.experimental.pallas import tpu as pltpu
from jax.experimental.pallas import tpu_sc as plsc

ROWS = 16384
COLS = 128
CHUNK = 128
NCH = ROWS // CHUNK
HALF = NCH // 2


def _sl(i):
    return pl.ds(i * CHUNK, CHUNK)


def _dual_pipeline(src_ref, dst_ref, bufA, spmem, sid, rsA, wsA, rsB, wsB):
    def rdA(i, slot):
        return pltpu.make_async_copy(src_ref.at[_sl(i)], bufA[slot],
                                     rsA[slot])

    def wrA(i, slot):
        return pltpu.make_async_copy(bufA[slot], dst_ref.at[_sl(i)],
                                     wsA[slot])

    def rdB(i, slot):
        return pltpu.make_async_copy(src_ref.at[_sl(i)],
                                     spmem.at[sid, slot], rsB[slot])

    def wrB(i, slot):
        return pltpu.make_async_copy(spmem.at[sid, slot],
                                     dst_ref.at[_sl(i)], wsB[slot])

    rdA(0, 0).start()
    rdB(HALF, 0).start()

    def body(g, carry):
        for gi in range(2):
            iA = g * 2 + gi
            iB = HALF + iA
            slot = gi
            nslot = (gi + 1) % 2

            rdA(iA, slot).wait()
            wrA(iA, slot).start()
            rdB(iB, slot).wait()
            wrB(iB, slot).start()

            @pl.when(iA >= 1)
            def _():
                wrA(iA - 1, nslot).wait()
                wrB(iB - 1, nslot).wait()

            @pl.when(iA + 1 < HALF)
            def _():
                rdA(iA + 1, nslot).start()
                rdB(iB + 1, nslot).start()
        return carry

    lax.fori_loop(0, HALF // 2, body, 0)
    lastslot = (HALF - 1) % 2
    wrA(HALF - 1, lastslot).wait()
    wrB(NCH - 1, lastslot).wait()


def _sc_body(x_hbm, src_hbm, out_hbm, b0, b1, spmem, *sems):
    c = lax.axis_index("c")
    s = lax.axis_index("s")
    w = s * 2 + c
    rsA = sems[0:2]
    wsA = sems[2:4]
    rsB = sems[4:6]
    wsB = sems[6:8]

    @pl.when(w == 0)
    def _():
        _dual_pipeline(src_hbm, out_hbm.at[0], (b0, b1), spmem, s,
                       rsA, wsA, rsB, wsB)

    @pl.when(w != 0)
    def _():
        _dual_pipeline(x_hbm.at[w], out_hbm.at[w], (b0, b1), spmem, s,
                       rsA, wsA, rsB, wsB)


def kernel(x, src):
    mesh = plsc.VectorSubcoreMesh(core_axis_name="c", subcore_axis_name="s")
    return pl.kernel(
        _sc_body,
        out_type=jax.ShapeDtypeStruct(x.shape, x.dtype),
        mesh=mesh,
        scratch_types=(
            [pltpu.VMEM((CHUNK, COLS), jnp.float32) for _ in range(2)]
            + [pltpu.VMEM_SHARED((16, 2, CHUNK, COLS), jnp.float32)]
            + [pltpu.SemaphoreType.DMA for _ in range(8)]
        ),
    )(x, src)


# speedup vs baseline: 1.1154x; 1.1154x over previous
import jax
import jax.numpy as jnp
from jax import lax
from jax.experimental import pallas as pl
from jax.experimental.pallas import tpu as pltpu
from jax.experimental.pallas import tpu_sc as plsc

N_ROWS = 32
ROWS = 16384
COLS = 128

CH = 4096
PER_ROW = ROWS // CH
NCH = (N_ROWS - 1) * PER_ROW
NBUF = 16
W = 8
K = NBUF - W
NGRP = -(-NCH // NBUF)


def _rd(x_hbm, buf, sem, i):
    r = 1 + i // PER_ROW
    sl = pl.ds((i % PER_ROW) * CH, CH)
    return pltpu.make_async_copy(x_hbm.at[r, sl], buf, sem)


def _wr(out_hbm, buf, sem, i):
    r = 1 + i // PER_ROW
    sl = pl.ds((i % PER_ROW) * CH, CH)
    return pltpu.make_async_copy(buf, out_hbm.at[r, sl], sem)


def _tc_body(x_hbm, out_hbm, *scratch):
    bufs = scratch[:NBUF]
    rsems = scratch[NBUF:2 * NBUF]
    wsems = scratch[2 * NBUF:]

    for j in range(K):
        _rd(x_hbm, bufs[j], rsems[j], j).start()

    def body(g, carry):
        for b in range(NBUF):
            i = g * NBUF + b

            @pl.when(i < NCH)
            def _():
                _rd(x_hbm, bufs[b], rsems[b], i).wait()
                _wr(out_hbm, bufs[b], wsems[b], i).start()

            bw = (b - W) % NBUF

            @pl.when(i >= W)
            def _():
                _wr(out_hbm, bufs[bw], wsems[bw], i - W).wait()

            br = (b + K) % NBUF

            @pl.when(i + K < NCH)
            def _():
                _rd(x_hbm, bufs[br], rsems[br], i + K).start()
        return carry

    lax.fori_loop(0, NGRP, body, 0)
    for i in range(NGRP * NBUF - W, NCH):
        b = i % NBUF
        _wr(out_hbm, bufs[b], wsems[b], i).wait()


_tc_pass_through = pl.pallas_call(
    _tc_body,
    out_shape=jax.ShapeDtypeStruct((N_ROWS, ROWS, COLS), jnp.float32),
    in_specs=[pl.BlockSpec(memory_space=pltpu.MemorySpace.HBM)],
    out_specs=pl.BlockSpec(memory_space=pltpu.MemorySpace.HBM),
    scratch_shapes=(
        [pltpu.VMEM((CH, COLS), jnp.float32) for _ in range(NBUF)]
        + [pltpu.SemaphoreType.DMA for _ in range(2 * NBUF)]
    ),
)


SC_STRIPE = ROWS // 32
SC_HALF = SC_STRIPE // 2


def _sc_slot0_body(src_hbm, out_hbm, b0, b1, s0, s1, t0, t1):
    c = lax.axis_index("c")
    s = lax.axis_index("s")
    w = s * 2 + c
    base = w * SC_STRIPE
    h0 = pl.ds(base, SC_HALF)
    h1 = pl.ds(base + SC_HALF, SC_HALF)
    pltpu.make_async_copy(src_hbm.at[h0], b0, s0).start()
    pltpu.make_async_copy(src_hbm.at[h1], b1, s1).start()
    pltpu.make_async_copy(src_hbm.at[h0], b0, s0).wait()
    pltpu.make_async_copy(b0, out_hbm.at[0, h0], t0).start()
    pltpu.make_async_copy(src_hbm.at[h1], b1, s1).wait()
    pltpu.make_async_copy(b1, out_hbm.at[0, h1], t1).start()
    pltpu.make_async_copy(b0, out_hbm.at[0, h0], t0).wait()
    pltpu.make_async_copy(b1, out_hbm.at[0, h1], t1).wait()


_sc_slot0 = pl.kernel(
    _sc_slot0_body,
    out_type=(),
    mesh=plsc.VectorSubcoreMesh(core_axis_name="c", subcore_axis_name="s"),
    scratch_types=(
        [pltpu.VMEM((SC_HALF, COLS), jnp.float32) for _ in range(2)]
        + [pltpu.SemaphoreType.DMA for _ in range(4)]
    ),
)


def kernel(x, src):
    out = _tc_pass_through(x)
    ref = jax.new_ref(out)
    _sc_slot0(src, ref)
    return ref[...]
